# Initial kernel scaffold; baseline (speedup 1.0000x reference)
import jax
import jax.numpy as jnp
from jax.experimental import pallas as pl


def kernel(x_snorna, x_disease, W1sd_rel, W1sd_root, b1sd, W1ds_rel, W1ds_root, b1ds, Wl2sd, Wr2sd, att2sd, b2sd, Wl2ds, Wr2ds, att2ds, b2ds, W3sd_rel, W3sd_root, b3sd, W3ds_rel, W3ds_root, b3ds, edge_index, edge_label_index):
    raise NotImplementedError("write your pallas kernel here")


# speedup vs baseline: 1.0337x; 1.0337x over previous
import functools

import jax
import jax.numpy as jnp
from jax.experimental import pallas as pl
from jax.experimental.pallas import tpu as pltpu

HEADS = 8


def _edge_dot_body(a_ref, b_ref, o_ref):
    o_ref[...] = jnp.sum(a_ref[...] * b_ref[...], axis=-1)


def _edge_dot(a, b):
    EL, F = a.shape
    BLK = 128
    n_rows = 800
    pad = n_rows * BLK - EL
    a3 = jnp.pad(a, ((0, pad), (0, 0))).reshape(n_rows, BLK, F)
    b3 = jnp.pad(b, ((0, pad), (0, 0))).reshape(n_rows, BLK, F)
    out = pl.pallas_call(
        _edge_dot_body,
        grid=(n_rows // 8,),
        in_specs=[
            pl.BlockSpec((8, BLK, F), lambda i: (i, 0, 0)),
            pl.BlockSpec((8, BLK, F), lambda i: (i, 0, 0)),
        ],
        out_specs=pl.BlockSpec((8, BLK), lambda i: (i, 0)),
        out_shape=jax.ShapeDtypeStruct((n_rows, BLK), jnp.float32),
    )(a3, b3)
    return out.reshape(-1)[:EL]


def _graph_conv(x_src, x_dst, src, dst, W_rel, b_rel, W_root):
    msg = jnp.take(x_src, src, axis=0)
    aggr = jax.ops.segment_sum(msg, dst, num_segments=x_dst.shape[0])
    return aggr @ W_rel + b_rel + x_dst @ W_root


def _gatv2_conv(x_src, x_dst, src, dst, Wl, Wr, att, bias):
    n_dst = x_dst.shape[0]
    xl = (x_src @ Wl).reshape(x_src.shape[0], HEADS, -1)
    xr = (x_dst @ Wr).reshape(n_dst, HEADS, -1)
    xj = jnp.take(xl, src, axis=0)
    xi = jnp.take(xr, dst, axis=0)
    e = jnp.sum(att * jax.nn.leaky_relu(xi + xj, 0.2), axis=-1)
    ex = jnp.exp(e)
    denom = jax.ops.segment_sum(ex, dst, num_segments=n_dst)
    alpha = ex / (jnp.take(denom, dst, axis=0) + 1e-16)
    out = jax.ops.segment_sum(alpha[..., None] * xj, dst, num_segments=n_dst)
    return out.mean(axis=1) + bias


def kernel(x_snorna, x_disease, W1sd_rel, W1sd_root, b1sd, W1ds_rel, W1ds_root, b1ds,
           Wl2sd, Wr2sd, att2sd, b2sd, Wl2ds, Wr2ds, att2ds, b2ds,
           W3sd_rel, W3sd_root, b3sd, W3ds_rel, W3ds_root, b3ds,
           edge_index, edge_label_index):
    s = edge_index[0].astype(jnp.int32)
    d = edge_index[1].astype(jnp.int32)
    xd = _graph_conv(x_snorna, x_disease, s, d, W1sd_rel, b1sd, W1sd_root)
    xs = _graph_conv(x_disease, x_snorna, d, s, W1ds_rel, b1ds, W1ds_root)
    xs = jax.nn.relu(xs)
    xd = jax.nn.relu(xd)
    nd = _gatv2_conv(xs, xd, s, d, Wl2sd, Wr2sd, att2sd, b2sd)
    ns = _gatv2_conv(xd, xs, d, s, Wl2ds, Wr2ds, att2ds, b2ds)
    xs = jax.nn.relu(ns)
    xd = jax.nn.relu(nd)
    nd = _graph_conv(xs, xd, s, d, W3sd_rel, b3sd, W3sd_root)
    ns = _graph_conv(xd, xs, d, s, W3ds_rel, b3ds, W3ds_root)
    ef_s = jnp.take(ns, edge_label_index[0].astype(jnp.int32), axis=0)
    ef_d = jnp.take(nd, edge_label_index[1].astype(jnp.int32), axis=0)
    return _edge_dot(ef_s, ef_d)


# speedup vs baseline: 7.1072x; 6.8752x over previous
import jax
import jax.numpy as jnp
from jax import lax
from jax.experimental import pallas as pl
from jax.experimental.pallas import tpu as pltpu
from jax.experimental.pallas import tpu_sc as plsc

HEADS = 8
N = 25000
NPAD = 25088
E = 600000
EPAD = 606208
EPT32 = EPAD // 32
EPT16 = EPAD // 16
EB = 512
RPT = NPAD // 16
EL = 100000
ELPAD = 100352
F32 = jnp.float32

_MESH = dict(core_axis_name="c", subcore_axis_name="s")
_CP = pltpu.CompilerParams(use_tc_tiling_on_sc=False, needs_layout_passes=False)


def _store_scalar_f32(ref, pos, s):
    lane = lax.iota(jnp.int32, 16)
    plsc.store_scatter(ref, [jnp.full((16,), pos, jnp.int32)],
                       jnp.full((16,), s, F32), mask=lane == 0)


def _zero_rows(ref, nrows, ncols):
    zv = jnp.zeros((16,), F32)

    @pl.loop(0, nrows)
    def _(i):
        for j in range(ncols // 16):
            ref[i, pl.ds(16 * j, 16)] = zv


def _make_segsum():
    def body(x_ref, src_ref, dst_ref, out_ref, idx_s, idx_d, rows, zrows, acc, sem):
        cid = lax.axis_index("c")
        sid = lax.axis_index("s")
        _zero_rows(zrows, EB, 32)
        r0 = sid * RPT
        base = sid * EPT16
        for cc in range(2):
            chunk = cid * 2 + cc
            coff = chunk * NPAD
            for t in range(3):
                pltpu.sync_copy(zrows, acc.at[pl.ds(r0 + t * EB, EB)])
            pltpu.sync_copy(zrows.at[pl.ds(0, 32)], acc.at[pl.ds(r0 + 3 * EB, 32)])
            plsc.subcore_barrier()

            @pl.loop(0, EPT16 // EB)
            def _(i):
                off = base + i * EB
                pltpu.sync_copy(src_ref.at[pl.ds(off, EB)], idx_s)
                pltpu.sync_copy(dst_ref.at[pl.ds(off, EB)], idx_d)
                for j in range(EB // 16):
                    sl = pl.ds(16 * j, 16)
                    idx_s[sl] = idx_s[sl] + coff
                pltpu.async_copy(x_ref.at[idx_s], rows, sem).wait()
                pltpu.sync_copy(rows, acc.at[idx_d], add=True)

            plsc.subcore_barrier()
            for t in range(3):
                pltpu.sync_copy(acc.at[pl.ds(r0 + t * EB, EB)], rows)
                pltpu.sync_copy(rows, out_ref.at[chunk].at[pl.ds(r0 + t * EB, EB)])
            pltpu.sync_copy(acc.at[pl.ds(r0 + 3 * EB, 32)], rows.at[pl.ds(0, 32)])
            pltpu.sync_copy(rows.at[pl.ds(0, 32)],
                            out_ref.at[chunk].at[pl.ds(r0 + 3 * EB, 32)])
            plsc.subcore_barrier()

    return pl.kernel(
        body,
        out_type=jax.ShapeDtypeStruct((4, NPAD, 32), F32),
        mesh=plsc.VectorSubcoreMesh(**_MESH),
        scratch_types=[
            pltpu.VMEM((EB,), jnp.int32),
            pltpu.VMEM((EB,), jnp.int32),
            pltpu.VMEM((EB, 32), F32),
            pltpu.VMEM((EB, 32), F32),
            pltpu.VMEM_SHARED((NPAD, 32), F32),
            pltpu.SemaphoreType.DMA,
        ],
        compiler_params=_CP,
    )


_segsum = _make_segsum()


B1 = 32
EU = 4


def _make_gat1():
    def body(xl_ref, xr_ref, src_ref, dst_ref, att_ref, ex_ref, den_ref,
             idx_s, idx_d, xlb, xrb, attb, exb1, exb2, denw, acc, sem):
        cid = lax.axis_index("c")
        sid = lax.axis_index("s")
        wid = sid * 2 + cid
        pltpu.sync_copy(att_ref, attb)
        _zero_rows(exb2, B1, 16)
        _zero_rows(denw, RPT // 2, 16)
        r0 = sid * RPT
        pltpu.sync_copy(denw, acc.at[pl.ds(r0, RPT // 2)])
        pltpu.sync_copy(denw, acc.at[pl.ds(r0 + RPT // 2, RPT // 2)])
        plsc.subcore_barrier()

        base = wid * EPT32

        @pl.loop(0, EPT32 // B1)
        def _(i):
            off = base + i * B1
            pltpu.sync_copy(src_ref.at[pl.ds(off, B1)], idx_s)
            pltpu.sync_copy(dst_ref.at[pl.ds(off, B1)], idx_d)
            d1 = pltpu.async_copy(xl_ref.at[idx_s], xlb, sem)
            d2 = pltpu.async_copy(xr_ref.at[idx_d], xrb, sem)
            d1.wait()
            d2.wait()

            @pl.loop(0, B1 // EU)
            def _(g):
                e0 = g * EU
                accs = [[jnp.zeros((16,), F32) for _ in range(HEADS)]
                        for _ in range(EU)]
                for jv in range(64):
                    h = jv // 8
                    av = attb[jv, :]
                    for u in range(EU):
                        z = xlb[e0 + u, pl.ds(16 * jv, 16)] + xrb[e0 + u, pl.ds(16 * jv, 16)]
                        z = jnp.maximum(z, 0.2 * z)
                        accs[u][h] = accs[u][h] + av * z
                for u in range(EU):
                    for h in range(HEADS):
                        s = jnp.sum(accs[u][h])
                        _store_scalar_f32(exb1, (e0 + u) * 8 + h, s)

            @pl.loop(0, B1 * 8 // 16)
            def _(m):
                v = jnp.exp(exb1[pl.ds(16 * m, 16)])
                exb1[pl.ds(16 * m, 16)] = v
                plsc.store_scatter(
                    exb2,
                    [(lax.iota(jnp.int32, 16) + 16 * m) // 8,
                     (lax.iota(jnp.int32, 16) + 16 * m) % 8],
                    v)

            pltpu.sync_copy(exb1, ex_ref.at[pl.ds(off * 8, B1 * 8)])
            pltpu.sync_copy(exb2, acc.at[idx_d], add=True)

        plsc.subcore_barrier()
        pltpu.sync_copy(acc.at[pl.ds(r0, RPT // 2)], denw)
        pltpu.sync_copy(denw, den_ref.at[cid].at[pl.ds(r0, RPT // 2)])
        pltpu.sync_copy(acc.at[pl.ds(r0 + RPT // 2, RPT // 2)], denw)
        pltpu.sync_copy(denw, den_ref.at[cid].at[pl.ds(r0 + RPT // 2, RPT // 2)])

    return pl.kernel(
        body,
        out_type=(jax.ShapeDtypeStruct((EPAD * 8,), F32),
                  jax.ShapeDtypeStruct((2, NPAD, 16), F32)),
        mesh=plsc.VectorSubcoreMesh(**_MESH),
        scratch_types=[
            pltpu.VMEM((B1,), jnp.int32),
            pltpu.VMEM((B1,), jnp.int32),
            pltpu.VMEM((B1, 1024), F32),
            pltpu.VMEM((B1, 1024), F32),
            pltpu.VMEM((64, 16), F32),
            pltpu.VMEM((B1 * 8,), F32),
            pltpu.VMEM((B1, 16), F32),
            pltpu.VMEM((RPT // 2, 16), F32),
            pltpu.VMEM_SHARED((NPAD, 16), F32),
            pltpu.SemaphoreType.DMA,
        ],
        compiler_params=_CP,
    )


_gat1 = _make_gat1()


def _make_gat2():
    def body(xl_ref, src_ref, dst_ref, ex_ref, out_ref,
             idx_s, idx_d, rows, zrows, exb, acc, sem):
        cid = lax.axis_index("c")
        sid = lax.axis_index("s")
        wid = sid * 2 + cid
        _zero_rows(zrows, EB, 32)
        r0 = sid * RPT
        base = wid * EPT32
        iota = lax.iota(jnp.int32, 16)
        for cc in range(16):
            chunk = cc * 2 + cid
            h = chunk // 4
            coff = chunk * NPAD
            for t in range(3):
                pltpu.sync_copy(zrows, acc.at[pl.ds(r0 + t * EB, EB)])
            pltpu.sync_copy(zrows.at[pl.ds(0, 32)], acc.at[pl.ds(r0 + 3 * EB, 32)])
            plsc.subcore_barrier()

            @pl.loop(0, EPT32 // EB)
            def _(i):
                off = base + i * EB
                pltpu.sync_copy(src_ref.at[pl.ds(off, EB)], idx_s)
                pltpu.sync_copy(dst_ref.at[pl.ds(off, EB)], idx_d)
                pltpu.sync_copy(ex_ref.at[pl.ds(off * 8, EB * 8)], exb)
                for j in range(EB // 16):
                    sl = pl.ds(16 * j, 16)
                    idx_s[sl] = idx_s[sl] + coff
                pltpu.async_copy(xl_ref.at[idx_s], rows, sem).wait()

                @pl.loop(0, EB, unroll=8)
                def _(e):
                    ev = plsc.load_gather(exb, [jnp.full((16,), e * 8 + h, jnp.int32)])
                    rows[e, pl.ds(0, 16)] = rows[e, pl.ds(0, 16)] * ev
                    rows[e, pl.ds(16, 16)] = rows[e, pl.ds(16, 16)] * ev

                pltpu.sync_copy(rows, acc.at[idx_d], add=True)

            plsc.subcore_barrier()
            for t in range(3):
                pltpu.sync_copy(acc.at[pl.ds(r0 + t * EB, EB)], rows)
                pltpu.sync_copy(rows, out_ref.at[chunk].at[pl.ds(r0 + t * EB, EB)])
            pltpu.sync_copy(acc.at[pl.ds(r0 + 3 * EB, 32)], rows.at[pl.ds(0, 32)])
            pltpu.sync_copy(rows.at[pl.ds(0, 32)],
                            out_ref.at[chunk].at[pl.ds(r0 + 3 * EB, 32)])
            plsc.subcore_barrier()

    return pl.kernel(
        body,
        out_type=jax.ShapeDtypeStruct((32, NPAD, 32), F32),
        mesh=plsc.VectorSubcoreMesh(**_MESH),
        scratch_types=[
            pltpu.VMEM((EB,), jnp.int32),
            pltpu.VMEM((EB,), jnp.int32),
            pltpu.VMEM((EB, 32), F32),
            pltpu.VMEM((EB, 32), F32),
            pltpu.VMEM((EB * 8,), F32),
            pltpu.VMEM_SHARED((NPAD, 32), F32),
            pltpu.SemaphoreType.DMA,
        ],
        compiler_params=_CP,
    )


_gat2 = _make_gat2()


CB = 448
CPT = ELPAD // 32


def _make_clf():
    def body(a_ref, b_ref, ia_ref, ib_ref, out_ref, idx_a, idx_b, ra, rb, ob, sem):
        cid = lax.axis_index("c")
        sid = lax.axis_index("s")
        wid = sid * 2 + cid
        base = wid * CPT

        @pl.loop(0, CPT // CB)
        def _(i):
            off = base + i * CB
            pltpu.sync_copy(ia_ref.at[pl.ds(off, CB)], idx_a)
            pltpu.sync_copy(ib_ref.at[pl.ds(off, CB)], idx_b)
            d1 = pltpu.async_copy(a_ref.at[idx_a], ra, sem)
            d2 = pltpu.async_copy(b_ref.at[idx_b], rb, sem)
            d1.wait()
            d2.wait()

            @pl.loop(0, CB, unroll=4)
            def _(e):
                acc = ra[e, pl.ds(0, 16)] * rb[e, pl.ds(0, 16)]
                for j in range(1, 4):
                    acc += ra[e, pl.ds(16 * j, 16)] * rb[e, pl.ds(16 * j, 16)]
                _store_scalar_f32(ob, e, jnp.sum(acc))

            pltpu.sync_copy(ob, out_ref.at[pl.ds(off, CB)])

    return pl.kernel(
        body,
        out_type=jax.ShapeDtypeStruct((ELPAD,), F32),
        mesh=plsc.VectorSubcoreMesh(**_MESH),
        scratch_types=[
            pltpu.VMEM((CB,), jnp.int32),
            pltpu.VMEM((CB,), jnp.int32),
            pltpu.VMEM((CB, 64), F32),
            pltpu.VMEM((CB, 64), F32),
            pltpu.VMEM((CB,), F32),
            pltpu.SemaphoreType.DMA,
        ],
        compiler_params=_CP,
    )


_clf = _make_clf()


def _l1mm_body(a_ref, x_ref, wrel_ref, wroot_ref, b_ref, o_ref):
    acc = jnp.dot(x_ref[...], wroot_ref[...], preferred_element_type=F32)
    for c in range(4):
        acc += jnp.dot(a_ref[c], wrel_ref[c], preferred_element_type=F32)
    o_ref[...] = jax.nn.relu(acc + b_ref[...])


def _l1mm(aggr, x, w_rel, b, w_root, h):
    BN = 512
    return pl.pallas_call(
        _l1mm_body,
        grid=(NPAD // BN,),
        in_specs=[
            pl.BlockSpec((4, BN, 32), lambda i: (0, i, 0)),
            pl.BlockSpec((BN, 128), lambda i: (i, 0)),
            pl.BlockSpec((4, 32, h), lambda i: (0, 0, 0)),
            pl.BlockSpec((128, h), lambda i: (0, 0)),
            pl.BlockSpec((1, h), lambda i: (0, 0)),
        ],
        out_specs=pl.BlockSpec((BN, h), lambda i: (i, 0)),
        out_shape=jax.ShapeDtypeStruct((NPAD, h), F32),
    )(aggr, x, w_rel.reshape(4, 32, h), w_root, b[None, :])


def _l3mm_body(a_ref, x_ref, wrel_ref, wroot_ref, b_ref, o_ref):
    acc = jnp.zeros(o_ref.shape, F32)
    for c in range(4):
        acc += jnp.dot(a_ref[c], wrel_ref[c], preferred_element_type=F32)
        acc += jnp.dot(x_ref[c], wroot_ref[c], preferred_element_type=F32)
    o_ref[...] = acc + b_ref[...]


def _l3mm(aggr, xc, w_rel, b, w_root):
    BN = 512
    h = 64
    return pl.pallas_call(
        _l3mm_body,
        grid=(NPAD // BN,),
        in_specs=[
            pl.BlockSpec((4, BN, 32), lambda i: (0, i, 0)),
            pl.BlockSpec((4, BN, 32), lambda i: (0, i, 0)),
            pl.BlockSpec((4, 32, h), lambda i: (0, 0, 0)),
            pl.BlockSpec((4, 32, h), lambda i: (0, 0, 0)),
            pl.BlockSpec((1, h), lambda i: (0, 0)),
        ],
        out_specs=pl.BlockSpec((BN, h), lambda i: (i, 0)),
        out_shape=jax.ShapeDtypeStruct((NPAD, h), F32),
    )(aggr, xc, w_rel.reshape(4, 32, h), w_root.reshape(4, 32, h), b[None, :])


def _proj_body(x_ref, w_ref, o1_ref, o2_ref):
    r = jnp.dot(x_ref[...], w_ref[...], preferred_element_type=F32)
    o1_ref[...] = r
    for j in range(4):
        o2_ref[j] = r[:, 32 * j:32 * j + 32]


def _proj(x, w):
    BN = 512
    return pl.pallas_call(
        _proj_body,
        grid=(8, NPAD // BN),
        in_specs=[
            pl.BlockSpec((BN, 128), lambda c, i: (i, 0)),
            pl.BlockSpec((128, 128), lambda c, i: (0, c)),
        ],
        out_specs=[
            pl.BlockSpec((BN, 128), lambda c, i: (i, c)),
            pl.BlockSpec((4, BN, 32), lambda c, i: (c, i, 0)),
        ],
        out_shape=[
            jax.ShapeDtypeStruct((NPAD, 1024), F32),
            jax.ShapeDtypeStruct((32, NPAD, 32), F32),
        ],
    )(x, w)


def _combine_body(num_ref, den_ref, b_ref, o_ref):
    den = den_ref[0, :, :8] + den_ref[1, :, :8] + 1e-16
    recip = 1.0 / den
    for j in range(4):
        acc = jnp.zeros(o_ref.shape[1:], F32)
        for h in range(HEADS):
            acc += num_ref[4 * h + j] * recip[:, h][:, None]
        o_ref[j] = jax.nn.relu(acc * 0.125 + b_ref[0, 32 * j:32 * j + 32])


def _combine(num, den2, bias):
    BN = 512
    return pl.pallas_call(
        _combine_body,
        grid=(NPAD // BN,),
        in_specs=[
            pl.BlockSpec((32, BN, 32), lambda i: (0, i, 0)),
            pl.BlockSpec((2, BN, 16), lambda i: (0, i, 0)),
            pl.BlockSpec((1, 128), lambda i: (0, 0)),
        ],
        out_specs=pl.BlockSpec((4, BN, 32), lambda i: (0, i, 0)),
        out_shape=jax.ShapeDtypeStruct((4, NPAD, 32), F32),
    )(num, den2, bias[None, :])


def _chunks4(x):
    xp = jnp.pad(x, ((0, NPAD - x.shape[0]), (0, 0)))
    return jnp.concatenate([xp[:, 32 * c:32 * c + 32] for c in range(4)], axis=0)


def kernel(x_snorna, x_disease, W1sd_rel, W1sd_root, b1sd, W1ds_rel, W1ds_root, b1ds,
           Wl2sd, Wr2sd, att2sd, b2sd, Wl2ds, Wr2ds, att2ds, b2ds,
           W3sd_rel, W3sd_root, b3sd, W3ds_rel, W3ds_root, b3ds,
           edge_index, edge_label_index):
    s = edge_index[0].astype(jnp.int32)
    d = edge_index[1].astype(jnp.int32)
    sp = jnp.concatenate([s, jnp.zeros((EPAD - E,), jnp.int32)])
    dp = jnp.concatenate([d, jnp.full((EPAD - E,), NPAD - 1, jnp.int32)])

    xs_pad = jnp.pad(x_snorna, ((0, NPAD - N), (0, 0)))
    xd_pad = jnp.pad(x_disease, ((0, NPAD - N), (0, 0)))

    aggr_d = _segsum(_chunks4(x_snorna), sp, dp)
    aggr_s = _segsum(_chunks4(x_disease), dp, sp)
    xd1 = _l1mm(aggr_d, xd_pad, W1sd_rel, b1sd, W1sd_root, 128)
    xs1 = _l1mm(aggr_s, xs_pad, W1ds_rel, b1ds, W1ds_root, 128)

    att_sd = att2sd.reshape(64, 16)
    att_ds = att2ds.reshape(64, 16)
    xl_sd_row, xl_sd_ch = _proj(xs1, Wl2sd)
    xr_sd_row, _ = _proj(xd1, Wr2sd)
    xl_ds_row, xl_ds_ch = _proj(xd1, Wl2ds)
    xr_ds_row, _ = _proj(xs1, Wr2ds)

    ex_sd, den_sd = _gat1(xl_sd_row, xr_sd_row, sp, dp, att_sd)
    ex_ds, den_ds = _gat1(xl_ds_row, xr_ds_row, dp, sp, att_ds)
    num_sd = _gat2(xl_sd_ch.reshape(32 * NPAD, 32), sp, dp, ex_sd)
    num_ds = _gat2(xl_ds_ch.reshape(32 * NPAD, 32), dp, sp, ex_ds)
    xd2c = _combine(num_sd, den_sd, b2sd)
    xs2c = _combine(num_ds, den_ds, b2ds)

    aggr_d3 = _segsum(xs2c.reshape(4 * NPAD, 32), sp, dp)
    aggr_s3 = _segsum(xd2c.reshape(4 * NPAD, 32), dp, sp)
    nd3 = _l3mm(aggr_d3, xd2c, W3sd_rel, b3sd, W3sd_root)
    ns3 = _l3mm(aggr_s3, xs2c, W3ds_rel, b3ds, W3ds_root)

    ia = jnp.concatenate([edge_label_index[0].astype(jnp.int32),
                          jnp.zeros((ELPAD - EL,), jnp.int32)])
    ib = jnp.concatenate([edge_label_index[1].astype(jnp.int32),
                          jnp.zeros((ELPAD - EL,), jnp.int32)])
    return _clf(ns3, nd3, ia, ib)[:EL]
